# Initial kernel scaffold; baseline (speedup 1.0000x reference)
import functools

import jax
import jax.numpy as jnp
from jax import lax
from jax.experimental import pallas as pl
from jax.experimental.pallas import tpu as pltpu
from jax.experimental.pallas import tpu_sc as plsc

N = 10000
E = 320000
D = 128
DE = 16
NG = 64

NC = 2
NS = 16
NW = NC * NS
CH = 128
CPW = -(-E // (NW * CH))
EPAD = CPW * NW * CH
NPAD = 10240
ZR = NPAD // NS
OR = NPAD // NS
PAD_DST = NPAD - 2

_MESH = dict(core_axis_name="c", subcore_axis_name="s")


def _sc_edge_tables(attrp, srcp, dstp, zrows):
    @functools.partial(
        pl.kernel,
        out_type=(
            jax.ShapeDtypeStruct((NC, NPAD, DE), jnp.float32),
            jax.ShapeDtypeStruct((NC, NPAD, DE), jnp.float32),
        ),
        mesh=plsc.VectorSubcoreMesh(**_MESH),
        scratch_types=[
            pltpu.VMEM((CH,), jnp.int32),
            pltpu.VMEM((CH,), jnp.int32),
            pltpu.VMEM((CH, DE), jnp.float32),
            pltpu.VMEM((CH, DE), jnp.float32),
            pltpu.VMEM_SHARED((NPAD, DE), jnp.float32),
            pltpu.VMEM_SHARED((NPAD, DE), jnp.float32),
            pltpu.SemaphoreType.DMA,
        ],
    )
    def k(attr_hbm, src_hbm, dst_hbm, zero_hbm, a_hbm, c_hbm,
          sidx, didx, rows, ones_v, tab_a, tab_c, sem):
        cid = lax.axis_index("c")
        sid = lax.axis_index("s")
        wid = cid * NS + sid

        @pl.loop(0, CH)
        def _(r):
            ones_v[r] = jnp.ones((DE,), jnp.float32)

        pltpu.sync_copy(zero_hbm, tab_a.at[pl.ds(sid * ZR, ZR)])
        pltpu.sync_copy(zero_hbm, tab_c.at[pl.ds(sid * ZR, ZR)])
        plsc.subcore_barrier()

        base0 = wid * (CPW * CH)

        @pl.loop(0, CPW)
        def _(i):
            base = base0 + i * CH
            pltpu.sync_copy(src_hbm.at[pl.ds(base, CH)], sidx)
            pltpu.sync_copy(dst_hbm.at[pl.ds(base, CH)], didx)
            pltpu.async_copy(attr_hbm.at[pl.ds(base, CH)], rows, sem).wait()
            pltpu.sync_copy(rows, tab_a.at[sidx], add=True)
            pltpu.sync_copy(rows, tab_a.at[didx], add=True)
            pltpu.sync_copy(ones_v, tab_c.at[didx], add=True)

        plsc.subcore_barrier()
        pltpu.sync_copy(tab_a.at[pl.ds(sid * OR, OR)],
                        a_hbm.at[cid, pl.ds(sid * OR, OR)])
        pltpu.sync_copy(tab_c.at[pl.ds(sid * OR, OR)],
                        c_hbm.at[cid, pl.ds(sid * OR, OR)])

    return k(attrp, srcp, dstp, zrows)


def _sc_gather_scatter(g, srcp, dstp, zrows):
    @functools.partial(
        pl.kernel,
        out_type=jax.ShapeDtypeStruct((NC, NPAD, D), jnp.float32),
        mesh=plsc.VectorSubcoreMesh(**_MESH),
        scratch_types=[
            pltpu.VMEM((CH,), jnp.int32),
            pltpu.VMEM((CH,), jnp.int32),
            pltpu.VMEM((CH, D), jnp.float32),
            pltpu.VMEM_SHARED((NPAD, D), jnp.float32),
            pltpu.SemaphoreType.DMA,
        ],
    )
    def k(g_hbm, src_hbm, dst_hbm, zero_hbm, out_hbm, sidx, didx, rows, tab, sem):
        cid = lax.axis_index("c")
        sid = lax.axis_index("s")
        wid = cid * NS + sid

        pltpu.sync_copy(zero_hbm, tab.at[pl.ds(sid * ZR, ZR)])
        plsc.subcore_barrier()

        base0 = wid * (CPW * CH)

        @pl.loop(0, CPW)
        def _(i):
            base = base0 + i * CH
            pltpu.sync_copy(src_hbm.at[pl.ds(base, CH)], sidx)
            pltpu.sync_copy(dst_hbm.at[pl.ds(base, CH)], didx)
            pltpu.async_copy(g_hbm.at[sidx], rows, sem).wait()
            pltpu.sync_copy(rows, tab.at[didx], add=True)

        plsc.subcore_barrier()
        pltpu.sync_copy(tab.at[pl.ds(sid * OR, OR)],
                        out_hbm.at[cid, pl.ds(sid * OR, OR)])

    return k(g, srcp, dstp, zrows)


def _tc_h1(x, W1):
    def body(x_ref, w_ref, o_ref):
        o_ref[...] = jnp.dot(x_ref[...], w_ref[...],
                             preferred_element_type=jnp.float32)

    return pl.pallas_call(
        body, out_shape=jax.ShapeDtypeStruct((N, D), jnp.float32),
    )(x, W1)


def _tc_prep(h1, A, C):
    def body(h_ref, a_ref, c_ref, g_ref, dinv_ref, e_ref):
        cnt = c_ref[0, :N, 0:1] + c_ref[1, :N, 0:1]
        dinv = lax.rsqrt(cnt + 1.0)
        g_ref[...] = h_ref[...] * dinv
        dinv_ref[...] = dinv
        e_ref[...] = a_ref[0, :N, :] + a_ref[1, :N, :]

    return pl.pallas_call(
        body,
        out_shape=(
            jax.ShapeDtypeStruct((N, D), jnp.float32),
            jax.ShapeDtypeStruct((N, 1), jnp.float32),
            jax.ShapeDtypeStruct((N, DE), jnp.float32),
        ),
    )(h1, A, C)


def _tc_mid(S1, g1, dinv, W2, b1):
    def body(s_ref, g_ref, d_ref, w_ref, b_ref, o_ref):
        dinv = d_ref[...]
        agg = s_ref[0, :N, :] + s_ref[1, :N, :] + g_ref[...]
        out1 = jnp.maximum(dinv * agg + b_ref[...], 0.0)
        h2 = jnp.dot(out1, w_ref[...], preferred_element_type=jnp.float32)
        o_ref[...] = h2 * dinv

    return pl.pallas_call(
        body, out_shape=jax.ShapeDtypeStruct((N, D), jnp.float32),
    )(S1, g1, dinv, W2, b1)


def _tc_final(S2, g2, dinv, b2, eagg, batchf, Wl, bl):
    def body(s_ref, g_ref, d_ref, b2_ref, e_ref, bt_ref, wl_ref, bl_ref, o_ref):
        agg = s_ref[0, :N, :] + s_ref[1, :N, :] + g_ref[...]
        out2 = d_ref[...] * agg + b2_ref[...]
        gid = lax.broadcasted_iota(jnp.float32, (NG, N), 0)
        oh = (gid == bt_ref[...]).astype(jnp.float32)
        sums_h = jnp.dot(oh, out2, preferred_element_type=jnp.float32)
        sums_e = jnp.dot(oh, e_ref[...], preferred_element_type=jnp.float32)
        counts = jnp.sum(oh, axis=1, keepdims=True)
        num = (jnp.dot(sums_h, wl_ref[:D, :], preferred_element_type=jnp.float32)
               + jnp.dot(sums_e, wl_ref[D:, :], preferred_element_type=jnp.float32))
        o_ref[...] = num / jnp.maximum(counts, 1.0) + bl_ref[...]

    return pl.pallas_call(
        body, out_shape=jax.ShapeDtypeStruct((NG, 1), jnp.float32),
    )(S2, g2, dinv, b2, eagg, batchf, Wl, bl)


def kernel(x, edge_index, edge_attr, batch, W1, b1, W2, b2, Wl, bl):
    src = edge_index[0].astype(jnp.int32)
    dst = edge_index[1].astype(jnp.int32)
    npad = EPAD - E
    srcp = jnp.concatenate([src, jnp.zeros((npad,), jnp.int32)])
    dstp = jnp.concatenate([dst, jnp.full((npad,), PAD_DST, jnp.int32)])
    attrp = jnp.concatenate(
        [edge_attr.astype(jnp.float32), jnp.zeros((npad, DE), jnp.float32)])
    z16 = jnp.zeros((ZR, DE), jnp.float32)
    z128 = jnp.zeros((ZR, D), jnp.float32)

    A, C = _sc_edge_tables(attrp, srcp, dstp, z16)
    h1 = _tc_h1(x, W1)
    g1, dinv, eagg = _tc_prep(h1, A, C)
    S1 = _sc_gather_scatter(g1, srcp, dstp, z128)
    g2 = _tc_mid(S1, g1, dinv, W2, b1.reshape(1, D))
    S2 = _sc_gather_scatter(g2, srcp, dstp, z128)
    batchf = batch.astype(jnp.float32).reshape(1, N)
    return _tc_final(S2, g2, dinv, b2.reshape(1, D), eagg, batchf,
                     Wl, bl.reshape(1, 1))


# speedup vs baseline: 7.7602x; 7.7602x over previous
import functools

import jax
import jax.numpy as jnp
from jax import lax
from jax.experimental import pallas as pl
from jax.experimental.pallas import tpu as pltpu
from jax.experimental.pallas import tpu_sc as plsc

N = 10000
E = 320000
D = 128
DE = 16
NG = 64

NC = 2
NS = 16
NW = NC * NS
CH = 128
CPW = -(-E // (NW * CH))
EPAD = CPW * NW * CH
NPAD = 10240
ZR = NPAD // NS
OR = NPAD // NS
PAD_DST = NPAD - 2

_MESH = dict(core_axis_name="c", subcore_axis_name="s")


def _sc_edge_tables(attrp, srcp, dstp, zrows):
    @functools.partial(
        pl.kernel,
        out_type=(
            jax.ShapeDtypeStruct((NC, NPAD, DE), jnp.float32),
            jax.ShapeDtypeStruct((NC, NPAD, DE), jnp.float32),
        ),
        mesh=plsc.VectorSubcoreMesh(**_MESH),
        scratch_types=[
            pltpu.VMEM((CH,), jnp.int32),
            pltpu.VMEM((CH,), jnp.int32),
            pltpu.VMEM((CH, DE), jnp.float32),
            pltpu.VMEM((CH, DE), jnp.float32),
            pltpu.VMEM_SHARED((NPAD, DE), jnp.float32),
            pltpu.VMEM_SHARED((NPAD, DE), jnp.float32),
            pltpu.SemaphoreType.DMA,
        ],
    )
    def k(attr_hbm, src_hbm, dst_hbm, zero_hbm, a_hbm, c_hbm,
          sidx, didx, rows, ones_v, tab_a, tab_c, sem):
        cid = lax.axis_index("c")
        sid = lax.axis_index("s")
        wid = cid * NS + sid

        @pl.loop(0, CH)
        def _(r):
            ones_v[r] = jnp.ones((DE,), jnp.float32)

        pltpu.sync_copy(zero_hbm, tab_a.at[pl.ds(sid * ZR, ZR)])
        pltpu.sync_copy(zero_hbm, tab_c.at[pl.ds(sid * ZR, ZR)])
        plsc.subcore_barrier()

        base0 = wid * (CPW * CH)

        @pl.loop(0, CPW)
        def _(i):
            base = base0 + i * CH
            pltpu.sync_copy(src_hbm.at[pl.ds(base, CH)], sidx)
            pltpu.sync_copy(dst_hbm.at[pl.ds(base, CH)], didx)
            pltpu.async_copy(attr_hbm.at[pl.ds(base, CH)], rows, sem).wait()
            pltpu.sync_copy(rows, tab_a.at[sidx], add=True)
            pltpu.sync_copy(rows, tab_a.at[didx], add=True)
            pltpu.sync_copy(ones_v, tab_c.at[didx], add=True)

        plsc.subcore_barrier()
        pltpu.sync_copy(tab_a.at[pl.ds(sid * OR, OR)],
                        a_hbm.at[cid, pl.ds(sid * OR, OR)])
        pltpu.sync_copy(tab_c.at[pl.ds(sid * OR, OR)],
                        c_hbm.at[cid, pl.ds(sid * OR, OR)])

    return k(attrp, srcp, dstp, zrows)


def _sc_gather_scatter(g, srcp, dstp, zrows):
    @functools.partial(
        pl.kernel,
        out_type=jax.ShapeDtypeStruct((NC, NPAD, D), jnp.float32),
        mesh=plsc.VectorSubcoreMesh(**_MESH),
        scratch_types=[
            pltpu.VMEM((CH,), jnp.int32),
            pltpu.VMEM((CH,), jnp.int32),
            pltpu.VMEM((CH, D), jnp.float32),
            pltpu.VMEM_SHARED((NPAD, D), jnp.float32),
            pltpu.SemaphoreType.DMA,
        ],
    )
    def k(g_hbm, src_hbm, dst_hbm, zero_hbm, out_hbm, sidx, didx, rows, tab, sem):
        cid = lax.axis_index("c")
        sid = lax.axis_index("s")
        wid = cid * NS + sid

        pltpu.sync_copy(zero_hbm, tab.at[pl.ds(sid * ZR, ZR)])
        plsc.subcore_barrier()

        base0 = wid * (CPW * CH)

        @pl.loop(0, CPW)
        def _(i):
            base = base0 + i * CH
            pltpu.sync_copy(src_hbm.at[pl.ds(base, CH)], sidx)
            pltpu.sync_copy(dst_hbm.at[pl.ds(base, CH)], didx)
            pltpu.async_copy(g_hbm.at[sidx], rows, sem).wait()
            pltpu.sync_copy(rows, tab.at[didx], add=True)

        plsc.subcore_barrier()
        pltpu.sync_copy(tab.at[pl.ds(sid * OR, OR)],
                        out_hbm.at[cid, pl.ds(sid * OR, OR)])

    return k(g, srcp, dstp, zrows)


def _tc_h1(x, W1):
    def body(x_ref, w_ref, o_ref):
        o_ref[...] = jnp.dot(x_ref[...], w_ref[...],
                             preferred_element_type=jnp.float32)

    return pl.pallas_call(
        body, out_shape=jax.ShapeDtypeStruct((N, D), jnp.float32),
    )(x, W1)


def _tc_prep(h1, A, C):
    def body(h_ref, a_ref, c_ref, g_ref, dinv_ref, e_ref):
        cnt = c_ref[0, :N, 0:1] + c_ref[1, :N, 0:1]
        dinv = lax.rsqrt(cnt + 1.0)
        g_ref[...] = h_ref[...] * dinv
        dinv_ref[...] = dinv
        e_ref[...] = a_ref[0, :N, :] + a_ref[1, :N, :]

    return pl.pallas_call(
        body,
        out_shape=(
            jax.ShapeDtypeStruct((N, D), jnp.float32),
            jax.ShapeDtypeStruct((N, 1), jnp.float32),
            jax.ShapeDtypeStruct((N, DE), jnp.float32),
        ),
    )(h1, A, C)


def _tc_mid(S1, g1, dinv, W2, b1):
    def body(s_ref, g_ref, d_ref, w_ref, b_ref, o_ref):
        dinv = d_ref[...]
        agg = s_ref[0, :N, :] + s_ref[1, :N, :] + g_ref[...]
        out1 = jnp.maximum(dinv * agg + b_ref[...], 0.0)
        h2 = jnp.dot(out1, w_ref[...], preferred_element_type=jnp.float32)
        o_ref[...] = h2 * dinv

    return pl.pallas_call(
        body, out_shape=jax.ShapeDtypeStruct((N, D), jnp.float32),
    )(S1, g1, dinv, W2, b1)


def _tc_final(S2, g2, dinv, b2, eagg, batchf, Wl, bl):
    def body(s_ref, g_ref, d_ref, b2_ref, e_ref, bt_ref, wl_ref, bl_ref, o_ref):
        agg = s_ref[0, :N, :] + s_ref[1, :N, :] + g_ref[...]
        out2 = d_ref[...] * agg + b2_ref[...]
        gid = lax.broadcasted_iota(jnp.int32, (NG, N), 0)
        oh = (gid == bt_ref[...]).astype(jnp.float32)
        sums_h = jnp.dot(oh, out2, preferred_element_type=jnp.float32)
        sums_e = jnp.dot(oh, e_ref[...], preferred_element_type=jnp.float32)
        counts = jnp.sum(oh, axis=1, keepdims=True)
        num = (jnp.dot(sums_h, wl_ref[:D, :], preferred_element_type=jnp.float32)
               + jnp.dot(sums_e, wl_ref[D:, :], preferred_element_type=jnp.float32))
        o_ref[...] = num / jnp.maximum(counts, 1.0) + bl_ref[...]

    return pl.pallas_call(
        body, out_shape=jax.ShapeDtypeStruct((NG, 1), jnp.float32),
    )(S2, g2, dinv, b2, eagg, batchf, Wl, bl)


def kernel(x, edge_index, edge_attr, batch, W1, b1, W2, b2, Wl, bl):
    src = edge_index[0].astype(jnp.int32)
    dst = edge_index[1].astype(jnp.int32)
    npad = EPAD - E
    srcp = jnp.concatenate([src, jnp.zeros((npad,), jnp.int32)])
    dstp = jnp.concatenate([dst, jnp.full((npad,), PAD_DST, jnp.int32)])
    attrp = jnp.concatenate(
        [edge_attr.astype(jnp.float32), jnp.zeros((npad, DE), jnp.float32)])
    z16 = jnp.zeros((ZR, DE), jnp.float32)
    z128 = jnp.zeros((ZR, D), jnp.float32)

    A, C = _sc_edge_tables(attrp, srcp, dstp, z16)
    h1 = _tc_h1(x, W1)
    g1, dinv, eagg = _tc_prep(h1, A, C)
    S1 = _sc_gather_scatter(g1, srcp, dstp, z128)
    g2 = _tc_mid(S1, g1, dinv, W2, b1.reshape(1, D))
    S2 = _sc_gather_scatter(g2, srcp, dstp, z128)
    batchf = batch.astype(jnp.int32).reshape(1, N)
    return _tc_final(S2, g2, dinv, b2.reshape(1, D), eagg, batchf,
                     Wl, bl.reshape(1, 1))
